# baseline (speedup 1.0000x reference)
import jax
import jax.numpy as jnp
from jax import lax
from jax.experimental import pallas as pl
from jax.experimental.pallas import tpu as pltpu
from jax.experimental.pallas import tpu_sc as plsc

N_RNA = 4000
N_ATAC = 6000
N_NODES = 10000
EMB = 256
TOPICS = 20
HG = 512
HM = 128
OUT = 10
E = 160000
BETA = 0.5

NC = 2
NS = 16
L = 16
NW = NC * NS
KA = 32
KB = 64
EPW = 5120
EPAD = NW * EPW
NCHA = EPW // KA
NCHB = EPW // KB
NPB = NCHB // 2
N_PAD = 10240
RPS = N_PAD // NS
ZROWS = 128


def _make_gat_sc(C):
    def body(comb, srcp, att, *rest):
        tabs = rest[0:C]
        denom_out = rest[C]
        num_out = rest[C + 1]
        (comball, srcall, dstva, dstvb, bufa, bufb, idxa, idxb, ebuf,
         wbuf, ldenom, attv, spacc, sema, semb, ssema, ssemb) = rest[C + 2:]

        core = lax.axis_index("c")
        sid = lax.axis_index("s")
        wid = core * NS + sid
        ebase = wid * EPW

        pltpu.sync_copy(att, attv)
        pltpu.sync_copy(comb.at[pl.ds(ebase * 2, EPW * 2)], comball)
        pltpu.sync_copy(srcp.at[pl.ds(ebase, EPW)], srcall)
        zeros16 = jnp.zeros((L,), jnp.float32)

        def zden(r, _):
            ldenom[pl.ds(r * L, L)] = zeros16
            return 0
        lax.fori_loop(0, N_PAD // L, zden, 0)

        bufs = (bufa, bufb)
        sems = (sema, semb)
        idxs = (idxa, idxb)

        def issue_a(k, c, b):
            for q in range(2 * KA // L):
                idxs[b][pl.ds(q * L, L)] = comball[
                    pl.ds(k * (2 * KA) + q * L, L)]
            pltpu.async_copy(tabs[c].at[idxs[b]], bufs[b], sems[b])

        def wait_a(k, c, b):
            pltpu.make_async_copy(
                tabs[c].at[idxs[b]], bufs[b], sems[b]).wait()

        issue_a(0, 0, 0)

        def chunk_a(k, _):
            base = ebase + k * KA
            for c in range(C):
                b = c % 2
                wait_a(k, c, b)
                if c + 1 < C:
                    issue_a(k, c + 1, (c + 1) % 2)
                else:
                    @pl.when(k < NCHA - 1)
                    def _():
                        issue_a(k + 1, 0, 0)
                buf = bufs[b]

                def jbody(j, _, c=c, buf=buf):
                    if c == 0:
                        av = zeros16
                    else:
                        av = ebuf[pl.ds(j * L, L)]
                    for t in range(8):
                        v = (buf[j, pl.ds(t * L, L)]
                             + buf[KA + j, pl.ds(t * L, L)])
                        z = jnp.maximum(v, 0.2 * v)
                        av = av + attv[pl.ds(c * 128 + t * L, L)] * z
                    ebuf[pl.ds(j * L, L)] = av
                    return 0
                lax.fori_loop(0, KA, jbody, 0)

            for q in range(KA // L):
                rvec = (lax.iota(jnp.int32, L) + q * L) * L
                ev = jnp.zeros((L,), jnp.float32)
                for t in range(L):
                    ev = ev + plsc.load_gather(ebuf, [rvec + t])
                gid = lax.iota(jnp.int32, L) + (base + q * L)
                w = jnp.where(gid < E, jnp.exp(ev), 0.0)
                wbuf[pl.ds(k * KA + q * L, L)] = w
                dvec = comball[pl.ds(k * (2 * KA) + KA + q * L, L)] - N_PAD
                plsc.addupdate_scatter(ldenom, [dvec], w)
            return 0
        lax.fori_loop(0, NCHA, chunk_a, 0)
        pltpu.sync_copy(ldenom, denom_out.at[pl.ds(wid * N_PAD, N_PAD)])

        def issue_b(m, c, b):
            for q in range(KB // L):
                idxs[b][pl.ds(q * L, L)] = srcall[pl.ds(m * KB + q * L, L)]
            pltpu.async_copy(tabs[c].at[idxs[b]], bufs[b], sems[b])

        def wait_b(m, c, b):
            pltpu.make_async_copy(
                tabs[c].at[idxs[b]], bufs[b], sems[b]).wait()

        dstvs = (dstva, dstvb)
        ssems = (ssema, ssemb)

        def process_b(m, b):
            for h in range(2):
                for q in range(KA // L):
                    s = comball[pl.ds((2 * m + h) * (2 * KA) + KA + q * L, L)]
                    dstvs[b][pl.ds(h * KA + q * L, L)] = s - N_PAD
            buf = bufs[b]

            def scale(qq, _, buf=buf):
                wv = wbuf[pl.ds(m * KB + qq * L, L)]
                for j in range(L):
                    w = wv[j]
                    r = qq * L + j
                    for t in range(8):
                        buf[r, pl.ds(t * L, L)] = buf[r, pl.ds(t * L, L)] * w
                return 0
            lax.fori_loop(0, KB // L, scale, 0)
            pltpu.async_copy(buf, spacc.at[dstvs[b]], ssems[b], add=True)

        def wait_s(b):
            pltpu.make_async_copy(
                bufs[b], spacc.at[dstvs[b]], ssems[b]).wait()

        for c in range(C):
            def zbuf(r, _):
                for t in range(8):
                    bufa[r, pl.ds(t * L, L)] = zeros16
                return 0
            lax.fori_loop(0, KB, zbuf, 0)
            for bb in range(RPS // KB):
                pltpu.sync_copy(
                    bufa, spacc.at[pl.ds(sid * RPS + bb * KB, KB)])
            plsc.subcore_barrier()

            issue_b(0, c, 0)

            def pair_b(p, _, c=c):
                m = 2 * p
                wait_b(m, c, 0)
                @pl.when(p > 0)
                def _():
                    wait_s(1)
                issue_b(m + 1, c, 1)
                process_b(m, 0)
                wait_b(m + 1, c, 1)
                wait_s(0)
                @pl.when(p < NPB - 1)
                def _():
                    issue_b(m + 2, c, 0)
                process_b(m + 1, 1)
                return 0
            lax.fori_loop(0, NPB, pair_b, 0)
            wait_s(1)
            plsc.subcore_barrier()
            for bb in range(RPS // ZROWS):
                r0 = sid * RPS + bb * ZROWS
                pltpu.sync_copy(spacc.at[pl.ds(r0, ZROWS)],
                                num_out.at[core, c, pl.ds(r0, ZROWS)])
            plsc.subcore_barrier()

    mesh = plsc.VectorSubcoreMesh(core_axis_name="c", subcore_axis_name="s",
                                  num_cores=NC, num_subcores=NS)
    return pl.kernel(
        body,
        out_type=(
            jax.ShapeDtypeStruct((NC * NS * N_PAD,), jnp.float32),
            jax.ShapeDtypeStruct((NC, C, N_PAD, 128), jnp.float32),
        ),
        mesh=mesh,
        compiler_params=pltpu.CompilerParams(needs_layout_passes=False),
        scratch_types=[
            pltpu.VMEM((EPW * 2,), jnp.int32),
            pltpu.VMEM((EPW,), jnp.int32),
            pltpu.VMEM((KB,), jnp.int32),
            pltpu.VMEM((KB,), jnp.int32),
            pltpu.VMEM((KB, 128), jnp.float32),
            pltpu.VMEM((KB, 128), jnp.float32),
            pltpu.VMEM((KB,), jnp.int32),
            pltpu.VMEM((KB,), jnp.int32),
            pltpu.VMEM((KA * L,), jnp.float32),
            pltpu.VMEM((EPW,), jnp.float32),
            pltpu.VMEM((N_PAD,), jnp.float32),
            pltpu.VMEM((C * 128,), jnp.float32),
            pltpu.VMEM_SHARED((N_PAD, 128), jnp.float32),
            pltpu.SemaphoreType.DMA,
            pltpu.SemaphoreType.DMA,
            pltpu.SemaphoreType.DMA,
            pltpu.SemaphoreType.DMA,
        ],
    )


_BLK = 400
_NBLK = N_NODES // _BLK
_BLKP = 512
_NBLKP = N_PAD // _BLKP


def _tc1_body(z0, cvec, wlr, *outs):
    feat = z0[...] * (1.0 + cvec[...])
    h = jnp.dot(feat, wlr[0], preferred_element_type=jnp.float32)
    for c in range(4):
        outs[c][...] = h[None, :, c * 128:(c + 1) * 128]


def _tc1(z0, cvec, wlr):
    return pl.pallas_call(
        _tc1_body,
        grid=(2, _NBLK),
        in_specs=[
            pl.BlockSpec((_BLK, EMB), lambda j, i: (i, 0)),
            pl.BlockSpec((_BLK, 1), lambda j, i: (i, 0)),
            pl.BlockSpec((1, EMB, HG), lambda j, i: (j, 0, 0)),
        ],
        out_specs=[pl.BlockSpec((1, _BLK, 128), lambda j, i: (j, i, 0))
                   for _ in range(4)],
        out_shape=[jax.ShapeDtypeStruct((2, N_PAD, 128), jnp.float32)
                   for _ in range(4)],
    )(z0, cvec, wlr)


def _tc2_body(num, den, b1, wlr, *outs):
    i = pl.program_id(1)
    n = num[...]
    s = n[0] + n[1]
    h = jnp.concatenate([s[c] for c in range(4)], axis=1)
    d = jnp.sum(den[:, :, pl.ds(i * _BLKP, _BLKP)], axis=(0, 1))
    h = h / (d[:, None] + 1e-16) + b1[...]
    h = jnp.where(h > 0, h, jnp.exp(jnp.minimum(h, 0.0)) - 1.0)
    hh = jnp.dot(h, wlr[0], preferred_element_type=jnp.float32)
    for c in range(2):
        outs[c][...] = hh[None, :, c * 128:(c + 1) * 128]


def _tc2(num, den, b1, wlr):
    return pl.pallas_call(
        _tc2_body,
        grid=(2, _NBLKP),
        in_specs=[
            pl.BlockSpec((NC, 4, _BLKP, 128), lambda j, i: (0, 0, i, 0)),
            pl.BlockSpec((NC, NS, N_PAD), lambda j, i: (0, 0, 0)),
            pl.BlockSpec((1, HG), lambda j, i: (0, 0)),
            pl.BlockSpec((1, HG, EMB), lambda j, i: (j, 0, 0)),
        ],
        out_specs=[pl.BlockSpec((1, _BLKP, 128), lambda j, i: (j, i, 0))
                   for _ in range(2)],
        out_shape=[jax.ShapeDtypeStruct((2, N_PAD, 128), jnp.float32)
                   for _ in range(2)],
    )(num, den, b1, wlr)


def _tc3a_body(num, den, b2, out):
    i = pl.program_id(0)
    n = num[...]
    s = n[0] + n[1]
    h = jnp.concatenate([s[c] for c in range(2)], axis=1)
    d = jnp.sum(den[:, :, pl.ds(i * _BLKP, _BLKP)], axis=(0, 1))
    h = h / (d[:, None] + 1e-16) + b2[...]
    ridx = lax.broadcasted_iota(jnp.int32, (_BLKP, EMB), 0) + i * _BLKP
    h = jnp.where(ridx < N_NODES, h, -jnp.inf)
    m = jnp.max(h, axis=0, keepdims=True)
    @pl.when(i == 0)
    def _():
        out[...] = m
    @pl.when(i > 0)
    def _():
        out[...] = jnp.maximum(out[...], m)


def _tc3a(num, den, b2):
    return pl.pallas_call(
        _tc3a_body,
        grid=(_NBLKP,),
        in_specs=[
            pl.BlockSpec((NC, 2, _BLKP, 128), lambda i: (0, 0, i, 0)),
            pl.BlockSpec((NC, NS, N_PAD), lambda i: (0, 0, 0)),
            pl.BlockSpec((1, EMB), lambda i: (0, 0)),
        ],
        out_specs=pl.BlockSpec((1, EMB), lambda i: (0, 0)),
        out_shape=jax.ShapeDtypeStruct((1, EMB), jnp.float32),
    )(num, den, b2)


def _tc3b_body(x_rna, x_atac, gpe, w_er, b_er, w_mr, b_mr, w_lr, b_lr,
               w_ea, b_ea, w_ma, b_ma, w_la, b_la, eps_r, eps_a,
               wm1t, wm1g, bm1, wm2, bm2, wc, bc, out):
    h_r = jnp.maximum(
        jnp.dot(x_rna[...], w_er[...], preferred_element_type=jnp.float32)
        + b_er[...], 0.0)
    mu_r = jnp.dot(h_r, w_mr[...]) + b_mr[...]
    lv_r = jnp.dot(h_r, w_lr[...]) + b_lr[...]
    h_a = jnp.maximum(
        jnp.dot(x_atac[...], w_ea[...], preferred_element_type=jnp.float32)
        + b_ea[...], 0.0)
    mu_a = jnp.dot(h_a, w_ma[...]) + b_ma[...]
    lv_a = jnp.dot(h_a, w_la[...]) + b_la[...]
    th_r = eps_r[...] * jnp.exp(0.5 * lv_r) + mu_r
    th_a = eps_a[...] * jnp.exp(0.5 * lv_a) + mu_a
    theta = th_r * (1.0 - BETA) + th_a * BETA
    o = (jnp.dot(theta, wm1t[...]) + jnp.dot(gpe[...], wm1g[...]) + bm1[...])
    o = jnp.maximum(o, 0.01 * o)
    o = jnp.dot(o, wm2[...]) + bm2[...]
    out[...] = jnp.dot(o, wc[...]) + bc[...]


def _tc3b(*args):
    return pl.pallas_call(
        _tc3b_body,
        out_shape=jax.ShapeDtypeStruct((1, OUT), jnp.float32),
    )(*args)


@jax.jit
def kernel(x_rna, x_atac, Z0, edge_index, W_enc_rna, b_enc_rna, W_mu_rna,
           b_mu_rna, W_lv_rna, b_lv_rna, W_enc_atac, b_enc_atac, W_mu_atac,
           b_mu_atac, W_lv_atac, b_lv_atac, Wl1, Wr1, att1, b1, Wl2, Wr2,
           att2, b2, Wm1, bm1, Wm2, bm2, Wc, bc):
    pad = jnp.zeros((EPAD - E,), jnp.int32)
    srcp = jnp.concatenate([edge_index[0], pad])
    dstp = jnp.concatenate([edge_index[1], pad])
    srcw = srcp.reshape(NW, NCHA, 1, KA)
    dstw = dstp.reshape(NW, NCHA, 1, KA) + N_PAD
    comb = jnp.concatenate([srcw, dstw], axis=2).reshape(-1)
    cvec = jnp.concatenate([x_atac, x_rna], axis=1).reshape(N_NODES, 1)
    kr = jax.random.key(42)
    eps_r = jax.random.normal(jax.random.fold_in(kr, 0), (1, TOPICS),
                              jnp.float32)
    eps_a = jax.random.normal(jax.random.fold_in(kr, 1), (1, TOPICS),
                              jnp.float32)

    t1 = _tc1(Z0, cvec, jnp.stack([Wl1, Wr1]))
    tabs1 = [t.reshape(2 * N_PAD, 128) for t in t1]
    den1, num1 = _make_gat_sc(4)(comb, srcp, att1, *tabs1)
    t2 = _tc2(num1, den1.reshape(NC, NS, N_PAD), b1.reshape(1, HG),
              jnp.stack([Wl2, Wr2]))
    tabs2 = [t.reshape(2 * N_PAD, 128) for t in t2]
    den2, num2 = _make_gat_sc(2)(comb, srcp, att2, *tabs2)
    gpe = _tc3a(num2, den2.reshape(NC, NS, N_PAD), b2.reshape(1, EMB))
    return _tc3b(
        x_rna, x_atac, gpe, W_enc_rna, b_enc_rna.reshape(1, EMB),
        W_mu_rna, b_mu_rna.reshape(1, TOPICS), W_lv_rna,
        b_lv_rna.reshape(1, TOPICS), W_enc_atac, b_enc_atac.reshape(1, EMB),
        W_mu_atac, b_mu_atac.reshape(1, TOPICS), W_lv_atac,
        b_lv_atac.reshape(1, TOPICS), eps_r, eps_a, Wm1[:TOPICS],
        Wm1[TOPICS:], bm1.reshape(1, HM), Wm2, bm2.reshape(1, 64), Wc,
        bc.reshape(1, OUT))


# speedup vs baseline: 1.2685x; 1.2685x over previous
import jax
import jax.numpy as jnp
from jax import lax
from jax.experimental import pallas as pl
from jax.experimental.pallas import tpu as pltpu
from jax.experimental.pallas import tpu_sc as plsc

N_RNA = 4000
N_ATAC = 6000
N_NODES = 10000
EMB = 256
TOPICS = 20
HG = 512
HM = 128
OUT = 10
E = 160000
BETA = 0.5

NC = 2
NS = 16
L = 16
NW = NC * NS
KA = 32
KB = 64
EPW = 5120
EPAD = NW * EPW
NCHA = EPW // KA
NCHB = EPW // KB
NPB = NCHB // 2
N_PAD = 10240
RPS = N_PAD // NS
ZROWS = 128


def _make_gat_sc(C):
    def body(comb, srcp, att, *rest):
        tabs = rest[0:C]
        denom_out = rest[C]
        num_out = rest[C + 1]
        (comball, srcall, dstv, bufa, bufb, idxa, idxb, ebuf, wbuf,
         ldenom, attv, spacc, sema, semb) = rest[C + 2:]

        core = lax.axis_index("c")
        sid = lax.axis_index("s")
        wid = core * NS + sid
        ebase = wid * EPW

        pltpu.sync_copy(att, attv)
        pltpu.sync_copy(comb.at[pl.ds(ebase * 2, EPW * 2)], comball)
        pltpu.sync_copy(srcp.at[pl.ds(ebase, EPW)], srcall)
        zeros16 = jnp.zeros((L,), jnp.float32)

        def zden(r, _):
            ldenom[pl.ds(r * L, L)] = zeros16
            return 0
        lax.fori_loop(0, N_PAD // L, zden, 0)

        bufs = (bufa, bufb)
        sems = (sema, semb)
        idxs = (idxa, idxb)

        def issue_a(k, c, b):
            for q in range(2 * KA // L):
                idxs[b][pl.ds(q * L, L)] = comball[
                    pl.ds(k * (2 * KA) + q * L, L)]
            pltpu.async_copy(tabs[c].at[idxs[b]], bufs[b], sems[b])

        def wait_a(k, c, b):
            pltpu.make_async_copy(
                tabs[c].at[idxs[b]], bufs[b], sems[b]).wait()

        issue_a(0, 0, 0)
        issue_a(0, 1, 1)

        def chunk_a(k, _):
            base = ebase + k * KA
            for c in range(C):
                b = c % 2
                wait_a(k, c, b)
                buf = bufs[b]

                def jbody(j, _, c=c, buf=buf):
                    if c == 0:
                        av = zeros16
                    else:
                        av = ebuf[pl.ds(j * L, L)]
                    for t in range(8):
                        v = (buf[j, pl.ds(t * L, L)]
                             + buf[KA + j, pl.ds(t * L, L)])
                        z = jnp.maximum(v, 0.2 * v)
                        av = av + attv[pl.ds(c * 128 + t * L, L)] * z
                    ebuf[pl.ds(j * L, L)] = av
                    return 0
                lax.fori_loop(0, KA, jbody, 0)
                if c + 2 < C:
                    issue_a(k, c + 2, b)
                else:
                    @pl.when(k < NCHA - 1)
                    def _(c=c, b=b):
                        issue_a(k + 1, c + 2 - C, b)

            for q in range(KA // L):
                rvec = (lax.iota(jnp.int32, L) + q * L) * L
                ev = jnp.zeros((L,), jnp.float32)
                for t in range(L):
                    ev = ev + plsc.load_gather(ebuf, [rvec + t])
                gid = lax.iota(jnp.int32, L) + (base + q * L)
                w = jnp.where(gid < E, jnp.exp(ev), 0.0)
                wbuf[pl.ds(k * KA + q * L, L)] = w
                dvec = comball[pl.ds(k * (2 * KA) + KA + q * L, L)] - N_PAD
                plsc.addupdate_scatter(ldenom, [dvec], w)
            return 0
        lax.fori_loop(0, NCHA, chunk_a, 0)
        pltpu.sync_copy(ldenom, denom_out.at[pl.ds(wid * N_PAD, N_PAD)])

        def issue_b(m, c, b):
            for q in range(KB // L):
                idxs[b][pl.ds(q * L, L)] = srcall[pl.ds(m * KB + q * L, L)]
            pltpu.async_copy(tabs[c].at[idxs[b]], bufs[b], sems[b])

        def wait_b(m, c, b):
            pltpu.make_async_copy(
                tabs[c].at[idxs[b]], bufs[b], sems[b]).wait()

        def process_b(m, b):
            for h in range(2):
                for q in range(KA // L):
                    s = comball[pl.ds((2 * m + h) * (2 * KA) + KA + q * L, L)]
                    dstv[pl.ds(h * KA + q * L, L)] = s - N_PAD
            buf = bufs[b]

            def scale(qq, _, buf=buf):
                wv = wbuf[pl.ds(m * KB + qq * L, L)]
                for j in range(L):
                    w = wv[j]
                    r = qq * L + j
                    for t in range(8):
                        buf[r, pl.ds(t * L, L)] = buf[r, pl.ds(t * L, L)] * w
                return 0
            lax.fori_loop(0, KB // L, scale, 0)
            pltpu.sync_copy(buf, spacc.at[dstv], add=True)

        for c in range(C):
            def zbuf(r, _):
                for t in range(8):
                    bufa[r, pl.ds(t * L, L)] = zeros16
                return 0
            lax.fori_loop(0, KB, zbuf, 0)
            for bb in range(RPS // KB):
                pltpu.sync_copy(
                    bufa, spacc.at[pl.ds(sid * RPS + bb * KB, KB)])
            plsc.subcore_barrier()

            issue_b(0, c, 0)
            issue_b(1, c, 1)

            def pair_b(p, _, c=c):
                m = 2 * p
                wait_b(m, c, 0)
                process_b(m, 0)
                @pl.when(p < NPB - 1)
                def _():
                    issue_b(m + 2, c, 0)
                wait_b(m + 1, c, 1)
                process_b(m + 1, 1)
                @pl.when(p < NPB - 1)
                def _():
                    issue_b(m + 3, c, 1)
                return 0
            lax.fori_loop(0, NPB, pair_b, 0)
            plsc.subcore_barrier()
            for bb in range(RPS // ZROWS):
                r0 = sid * RPS + bb * ZROWS
                pltpu.sync_copy(spacc.at[pl.ds(r0, ZROWS)],
                                num_out.at[core, c, pl.ds(r0, ZROWS)])
            plsc.subcore_barrier()

    mesh = plsc.VectorSubcoreMesh(core_axis_name="c", subcore_axis_name="s",
                                  num_cores=NC, num_subcores=NS)
    return pl.kernel(
        body,
        out_type=(
            jax.ShapeDtypeStruct((NC * NS * N_PAD,), jnp.float32),
            jax.ShapeDtypeStruct((NC, C, N_PAD, 128), jnp.float32),
        ),
        mesh=mesh,
        compiler_params=pltpu.CompilerParams(needs_layout_passes=False),
        scratch_types=[
            pltpu.VMEM((EPW * 2,), jnp.int32),
            pltpu.VMEM((EPW,), jnp.int32),
            pltpu.VMEM((KB,), jnp.int32),
            pltpu.VMEM((KB, 128), jnp.float32),
            pltpu.VMEM((KB, 128), jnp.float32),
            pltpu.VMEM((KB,), jnp.int32),
            pltpu.VMEM((KB,), jnp.int32),
            pltpu.VMEM((KA * L,), jnp.float32),
            pltpu.VMEM((EPW,), jnp.float32),
            pltpu.VMEM((N_PAD,), jnp.float32),
            pltpu.VMEM((C * 128,), jnp.float32),
            pltpu.VMEM_SHARED((N_PAD, 128), jnp.float32),
            pltpu.SemaphoreType.DMA,
            pltpu.SemaphoreType.DMA,
        ],
    )


_BLK = 400
_NBLK = N_NODES // _BLK
_BLKP = 512
_NBLKP = N_PAD // _BLKP


def _tc1_body(z0, cvec, wlr, *outs):
    feat = z0[...] * (1.0 + cvec[...])
    h = jnp.dot(feat, wlr[0], preferred_element_type=jnp.float32)
    for c in range(4):
        outs[c][...] = h[None, :, c * 128:(c + 1) * 128]


def _tc1(z0, cvec, wlr):
    return pl.pallas_call(
        _tc1_body,
        grid=(2, _NBLK),
        in_specs=[
            pl.BlockSpec((_BLK, EMB), lambda j, i: (i, 0)),
            pl.BlockSpec((_BLK, 1), lambda j, i: (i, 0)),
            pl.BlockSpec((1, EMB, HG), lambda j, i: (j, 0, 0)),
        ],
        out_specs=[pl.BlockSpec((1, _BLK, 128), lambda j, i: (j, i, 0))
                   for _ in range(4)],
        out_shape=[jax.ShapeDtypeStruct((2, N_PAD, 128), jnp.float32)
                   for _ in range(4)],
    )(z0, cvec, wlr)


def _tc2_body(num, den, b1, wlr, *outs):
    i = pl.program_id(1)
    n = num[...]
    s = n[0] + n[1]
    h = jnp.concatenate([s[c] for c in range(4)], axis=1)
    d = jnp.sum(den[:, :, pl.ds(i * _BLKP, _BLKP)], axis=(0, 1))
    h = h / (d[:, None] + 1e-16) + b1[...]
    h = jnp.where(h > 0, h, jnp.exp(jnp.minimum(h, 0.0)) - 1.0)
    hh = jnp.dot(h, wlr[0], preferred_element_type=jnp.float32)
    for c in range(2):
        outs[c][...] = hh[None, :, c * 128:(c + 1) * 128]


def _tc2(num, den, b1, wlr):
    return pl.pallas_call(
        _tc2_body,
        grid=(2, _NBLKP),
        in_specs=[
            pl.BlockSpec((NC, 4, _BLKP, 128), lambda j, i: (0, 0, i, 0)),
            pl.BlockSpec((NC, NS, N_PAD), lambda j, i: (0, 0, 0)),
            pl.BlockSpec((1, HG), lambda j, i: (0, 0)),
            pl.BlockSpec((1, HG, EMB), lambda j, i: (j, 0, 0)),
        ],
        out_specs=[pl.BlockSpec((1, _BLKP, 128), lambda j, i: (j, i, 0))
                   for _ in range(2)],
        out_shape=[jax.ShapeDtypeStruct((2, N_PAD, 128), jnp.float32)
                   for _ in range(2)],
    )(num, den, b1, wlr)


def _tc3a_body(num, den, b2, out):
    i = pl.program_id(0)
    n = num[...]
    s = n[0] + n[1]
    h = jnp.concatenate([s[c] for c in range(2)], axis=1)
    d = jnp.sum(den[:, :, pl.ds(i * _BLKP, _BLKP)], axis=(0, 1))
    h = h / (d[:, None] + 1e-16) + b2[...]
    ridx = lax.broadcasted_iota(jnp.int32, (_BLKP, EMB), 0) + i * _BLKP
    h = jnp.where(ridx < N_NODES, h, -jnp.inf)
    m = jnp.max(h, axis=0, keepdims=True)
    @pl.when(i == 0)
    def _():
        out[...] = m
    @pl.when(i > 0)
    def _():
        out[...] = jnp.maximum(out[...], m)


def _tc3a(num, den, b2):
    return pl.pallas_call(
        _tc3a_body,
        grid=(_NBLKP,),
        in_specs=[
            pl.BlockSpec((NC, 2, _BLKP, 128), lambda i: (0, 0, i, 0)),
            pl.BlockSpec((NC, NS, N_PAD), lambda i: (0, 0, 0)),
            pl.BlockSpec((1, EMB), lambda i: (0, 0)),
        ],
        out_specs=pl.BlockSpec((1, EMB), lambda i: (0, 0)),
        out_shape=jax.ShapeDtypeStruct((1, EMB), jnp.float32),
    )(num, den, b2)


def _tc3b_body(x_rna, x_atac, gpe, w_er, b_er, w_mr, b_mr, w_lr, b_lr,
               w_ea, b_ea, w_ma, b_ma, w_la, b_la, eps_r, eps_a,
               wm1t, wm1g, bm1, wm2, bm2, wc, bc, out):
    h_r = jnp.maximum(
        jnp.dot(x_rna[...], w_er[...], preferred_element_type=jnp.float32)
        + b_er[...], 0.0)
    mu_r = jnp.dot(h_r, w_mr[...]) + b_mr[...]
    lv_r = jnp.dot(h_r, w_lr[...]) + b_lr[...]
    h_a = jnp.maximum(
        jnp.dot(x_atac[...], w_ea[...], preferred_element_type=jnp.float32)
        + b_ea[...], 0.0)
    mu_a = jnp.dot(h_a, w_ma[...]) + b_ma[...]
    lv_a = jnp.dot(h_a, w_la[...]) + b_la[...]
    th_r = eps_r[...] * jnp.exp(0.5 * lv_r) + mu_r
    th_a = eps_a[...] * jnp.exp(0.5 * lv_a) + mu_a
    theta = th_r * (1.0 - BETA) + th_a * BETA
    o = (jnp.dot(theta, wm1t[...]) + jnp.dot(gpe[...], wm1g[...]) + bm1[...])
    o = jnp.maximum(o, 0.01 * o)
    o = jnp.dot(o, wm2[...]) + bm2[...]
    out[...] = jnp.dot(o, wc[...]) + bc[...]


def _tc3b(*args):
    return pl.pallas_call(
        _tc3b_body,
        out_shape=jax.ShapeDtypeStruct((1, OUT), jnp.float32),
    )(*args)


@jax.jit
def kernel(x_rna, x_atac, Z0, edge_index, W_enc_rna, b_enc_rna, W_mu_rna,
           b_mu_rna, W_lv_rna, b_lv_rna, W_enc_atac, b_enc_atac, W_mu_atac,
           b_mu_atac, W_lv_atac, b_lv_atac, Wl1, Wr1, att1, b1, Wl2, Wr2,
           att2, b2, Wm1, bm1, Wm2, bm2, Wc, bc):
    pad = jnp.zeros((EPAD - E,), jnp.int32)
    srcp = jnp.concatenate([edge_index[0], pad])
    dstp = jnp.concatenate([edge_index[1], pad])
    srcw = srcp.reshape(NW, NCHA, 1, KA)
    dstw = dstp.reshape(NW, NCHA, 1, KA) + N_PAD
    comb = jnp.concatenate([srcw, dstw], axis=2).reshape(-1)
    cvec = jnp.concatenate([x_atac, x_rna], axis=1).reshape(N_NODES, 1)
    kr = jax.random.key(42)
    eps_r = jax.random.normal(jax.random.fold_in(kr, 0), (1, TOPICS),
                              jnp.float32)
    eps_a = jax.random.normal(jax.random.fold_in(kr, 1), (1, TOPICS),
                              jnp.float32)

    t1 = _tc1(Z0, cvec, jnp.stack([Wl1, Wr1]))
    tabs1 = [t.reshape(2 * N_PAD, 128) for t in t1]
    den1, num1 = _make_gat_sc(4)(comb, srcp, att1, *tabs1)
    t2 = _tc2(num1, den1.reshape(NC, NS, N_PAD), b1.reshape(1, HG),
              jnp.stack([Wl2, Wr2]))
    tabs2 = [t.reshape(2 * N_PAD, 128) for t in t2]
    den2, num2 = _make_gat_sc(2)(comb, srcp, att2, *tabs2)
    gpe = _tc3a(num2, den2.reshape(NC, NS, N_PAD), b2.reshape(1, EMB))
    return _tc3b(
        x_rna, x_atac, gpe, W_enc_rna, b_enc_rna.reshape(1, EMB),
        W_mu_rna, b_mu_rna.reshape(1, TOPICS), W_lv_rna,
        b_lv_rna.reshape(1, TOPICS), W_enc_atac, b_enc_atac.reshape(1, EMB),
        W_mu_atac, b_mu_atac.reshape(1, TOPICS), W_lv_atac,
        b_lv_atac.reshape(1, TOPICS), eps_r, eps_a, Wm1[:TOPICS],
        Wm1[TOPICS:], bm1.reshape(1, HM), Wm2, bm2.reshape(1, 64), Wc,
        bc.reshape(1, OUT))
